# baseline (speedup 1.0000x reference)
import functools

import jax
import jax.numpy as jnp
from jax.experimental import pallas as pl
from jax.experimental.pallas import tpu as pltpu

_TOPK = 16
_TEMP = 10.0
_BK = 1024
_GROUP = 128
_IMAX = 2147483647


def _topk_kernel(q_ref, k_ref, vals_ref, idxs_ref,
                 s_ref, v_ref, i_ref, gv_ref, *, nk, kdim):
    kb = pl.program_id(0)
    q = q_ref.shape[0]
    ngrp = _BK // _GROUP
    neg_inf = jnp.float32(-jnp.inf)

    @pl.when(kb == 0)
    def _init():
        v_ref[...] = jnp.full((q, _TOPK), neg_inf, jnp.float32)
        i_ref[...] = jnp.full((q, _TOPK), _IMAX, jnp.int32)

    s = jax.lax.dot_general(
        q_ref[...], k_ref[...], (((1,), (1,)), ((), ())),
        preferred_element_type=jnp.float32) * _TEMP
    s_ref[...] = s

    @pl.when(kb == nk - 1)
    def _mask_tail():
        rem = kdim - kb * _BK
        col = jax.lax.broadcasted_iota(jnp.int32, (q, _BK), 1)
        s_ref[...] = jnp.where(col < rem, s_ref[...], neg_inf)

    tio = jax.lax.broadcasted_iota(jnp.int32, (q, _TOPK), 1)
    lio = jax.lax.broadcasted_iota(jnp.int32, (q, _GROUP), 1)

    def _group_max():
        gv_ref[...] = jnp.concatenate(
            [jnp.max(s_ref[:, g * _GROUP:(g + 1) * _GROUP], axis=1,
                     keepdims=True) for g in range(ngrp)], axis=1)

    def _maybe_improves():
        return jnp.any(gv_ref[...] >= v_ref[:, _TOPK - 1:_TOPK])

    _group_max()

    def _cond(cont):
        return cont

    def _body(cont):
        gv = gv_ref[...]
        for g in range(ngrp):
            m = gv[:, g:g + 1]
            sl = slice(g * _GROUP, (g + 1) * _GROUP)
            sg = s_ref[:, sl]
            am = jnp.min(jnp.where(sg == m, lio, _GROUP), axis=1,
                         keepdims=True)
            ci = am + (kb * _BK + g * _GROUP)
            s_ref[:, sl] = jnp.where(lio == am, neg_inf, sg)
            v = v_ref[...]
            i = i_ref[...]
            cnt = jnp.sum(
                ((v > m) | ((v == m) & (i < ci))).astype(jnp.int32),
                axis=1, keepdims=True)
            vsh = jnp.concatenate([v[:, :1], v[:, :_TOPK - 1]], axis=1)
            ish = jnp.concatenate([i[:, :1], i[:, :_TOPK - 1]], axis=1)
            v_ref[...] = jnp.where(tio < cnt, v,
                                   jnp.where(tio == cnt, m, vsh))
            i_ref[...] = jnp.where(tio < cnt, i,
                                   jnp.where(tio == cnt, ci, ish))
        _group_max()
        return _maybe_improves()

    jax.lax.while_loop(_cond, _body, _maybe_improves())

    @pl.when(kb == nk - 1)
    def _finalize():
        vals_ref[...] = v_ref[...]
        idxs_ref[...] = i_ref[...]


def kernel(queries, keys):
    qn, d = queries.shape
    kdim = keys.shape[0]
    nk = pl.cdiv(kdim, _BK)
    ngrp = _BK // _GROUP
    vals, idxs = pl.pallas_call(
        functools.partial(_topk_kernel, nk=nk, kdim=kdim),
        grid=(nk,),
        in_specs=[
            pl.BlockSpec((qn, d), lambda kb: (0, 0)),
            pl.BlockSpec((_BK, d), lambda kb: (kb, 0)),
        ],
        out_specs=[
            pl.BlockSpec((qn, _TOPK), lambda kb: (0, 0)),
            pl.BlockSpec((qn, _TOPK), lambda kb: (0, 0)),
        ],
        out_shape=[
            jax.ShapeDtypeStruct((qn, _TOPK), jnp.float32),
            jax.ShapeDtypeStruct((qn, _TOPK), jnp.int32),
        ],
        scratch_shapes=[
            pltpu.VMEM((qn, _BK), jnp.float32),
            pltpu.VMEM((qn, _TOPK), jnp.float32),
            pltpu.VMEM((qn, _TOPK), jnp.int32),
            pltpu.VMEM((qn, ngrp), jnp.float32),
        ],
        compiler_params=pltpu.CompilerParams(
            dimension_semantics=("arbitrary",)),
    )(queries, keys)
    return vals, idxs


# speedup vs baseline: 1.0239x; 1.0239x over previous
import functools

import jax
import jax.numpy as jnp
from jax.experimental import pallas as pl
from jax.experimental.pallas import tpu as pltpu

_TOPK = 16
_TEMP = 10.0
_BK = 2048
_GROUP = 128
_IMAX = 2147483647


def _topk_kernel(q_ref, k_ref, vals_ref, idxs_ref,
                 s_ref, v_ref, i_ref, gv_ref, *, nk, kdim):
    kb = pl.program_id(0)
    q = q_ref.shape[0]
    ngrp = _BK // _GROUP
    neg_inf = jnp.float32(-jnp.inf)

    @pl.when(kb == 0)
    def _init():
        v_ref[...] = jnp.full((q, _TOPK), neg_inf, jnp.float32)
        i_ref[...] = jnp.full((q, _TOPK), _IMAX, jnp.int32)

    s = jax.lax.dot_general(
        q_ref[...], k_ref[...], (((1,), (1,)), ((), ())),
        preferred_element_type=jnp.float32) * _TEMP
    s_ref[...] = s

    @pl.when(kb == nk - 1)
    def _mask_tail():
        rem = kdim - kb * _BK
        col = jax.lax.broadcasted_iota(jnp.int32, (q, _BK), 1)
        s_ref[...] = jnp.where(col < rem, s_ref[...], neg_inf)

    tio = jax.lax.broadcasted_iota(jnp.int32, (q, _TOPK), 1)
    lio = jax.lax.broadcasted_iota(jnp.int32, (q, _GROUP), 1)

    def _group_max():
        gv_ref[...] = jnp.concatenate(
            [jnp.max(s_ref[:, g * _GROUP:(g + 1) * _GROUP], axis=1,
                     keepdims=True) for g in range(ngrp)], axis=1)

    def _maybe_improves():
        return jnp.any(gv_ref[...] >= v_ref[:, _TOPK - 1:_TOPK])

    _group_max()

    def _cond(cont):
        return cont

    def _body(cont):
        gv = gv_ref[...]
        for g in range(ngrp):
            m = gv[:, g:g + 1]
            sl = slice(g * _GROUP, (g + 1) * _GROUP)
            sg = s_ref[:, sl]
            am = jnp.min(jnp.where(sg == m, lio, _GROUP), axis=1,
                         keepdims=True)
            ci = am + (kb * _BK + g * _GROUP)
            s_ref[:, sl] = jnp.where(lio == am, neg_inf, sg)
            v = v_ref[...]
            i = i_ref[...]
            cnt = jnp.sum(
                ((v > m) | ((v == m) & (i < ci))).astype(jnp.int32),
                axis=1, keepdims=True)
            vsh = jnp.concatenate([v[:, :1], v[:, :_TOPK - 1]], axis=1)
            ish = jnp.concatenate([i[:, :1], i[:, :_TOPK - 1]], axis=1)
            v_ref[...] = jnp.where(tio < cnt, v,
                                   jnp.where(tio == cnt, m, vsh))
            i_ref[...] = jnp.where(tio < cnt, i,
                                   jnp.where(tio == cnt, ci, ish))
        _group_max()
        return _maybe_improves()

    jax.lax.while_loop(_cond, _body, _maybe_improves())

    @pl.when(kb == nk - 1)
    def _finalize():
        vals_ref[...] = v_ref[...]
        idxs_ref[...] = i_ref[...]


def kernel(queries, keys):
    qn, d = queries.shape
    kdim = keys.shape[0]
    nk = pl.cdiv(kdim, _BK)
    ngrp = _BK // _GROUP
    vals, idxs = pl.pallas_call(
        functools.partial(_topk_kernel, nk=nk, kdim=kdim),
        grid=(nk,),
        in_specs=[
            pl.BlockSpec((qn, d), lambda kb: (0, 0)),
            pl.BlockSpec((_BK, d), lambda kb: (kb, 0)),
        ],
        out_specs=[
            pl.BlockSpec((qn, _TOPK), lambda kb: (0, 0)),
            pl.BlockSpec((qn, _TOPK), lambda kb: (0, 0)),
        ],
        out_shape=[
            jax.ShapeDtypeStruct((qn, _TOPK), jnp.float32),
            jax.ShapeDtypeStruct((qn, _TOPK), jnp.int32),
        ],
        scratch_shapes=[
            pltpu.VMEM((qn, _BK), jnp.float32),
            pltpu.VMEM((qn, _TOPK), jnp.float32),
            pltpu.VMEM((qn, _TOPK), jnp.int32),
            pltpu.VMEM((qn, ngrp), jnp.float32),
        ],
        compiler_params=pltpu.CompilerParams(
            dimension_semantics=("arbitrary",)),
    )(queries, keys)
    return vals, idxs


# speedup vs baseline: 2.3384x; 2.2840x over previous
import functools

import jax
import jax.numpy as jnp
from jax.experimental import pallas as pl
from jax.experimental.pallas import tpu as pltpu

_TOPK = 16
_TEMP = 10.0
_BK = 2048
_GROUP = 512
_IMAX = 2147483647


def _topk_kernel(q_ref, k_ref, vals_ref, idxs_ref,
                 s_ref, v_ref, i_ref, gv_ref, *, nk, kdim):
    kb = pl.program_id(0)
    q = q_ref.shape[0]
    ngrp = _BK // _GROUP
    neg_inf = jnp.float32(-jnp.inf)

    @pl.when(kb == 0)
    def _init():
        v_ref[...] = jnp.full((q, _TOPK), neg_inf, jnp.float32)
        i_ref[...] = jnp.full((q, _TOPK), _IMAX, jnp.int32)

    s = jax.lax.dot_general(
        q_ref[...], k_ref[...], (((1,), (1,)), ((), ())),
        preferred_element_type=jnp.float32) * _TEMP
    s_ref[...] = s

    @pl.when(kb == nk - 1)
    def _mask_tail():
        rem = kdim - kb * _BK
        col = jax.lax.broadcasted_iota(jnp.int32, (q, _BK), 1)
        s_ref[...] = jnp.where(col < rem, s_ref[...], neg_inf)

    tio = jax.lax.broadcasted_iota(jnp.int32, (q, _TOPK), 1)
    lio = jax.lax.broadcasted_iota(jnp.int32, (q, _GROUP), 1)

    def _group_max():
        gv_ref[...] = jnp.concatenate(
            [jnp.max(s_ref[:, g * _GROUP:(g + 1) * _GROUP], axis=1,
                     keepdims=True) for g in range(ngrp)], axis=1)

    def _maybe_improves():
        return jnp.any(gv_ref[...] >= v_ref[:, _TOPK - 1:_TOPK])

    _group_max()

    def _cond(cont):
        return cont

    def _body(cont):
        gv = gv_ref[...]
        for g in range(ngrp):
            m = gv[:, g:g + 1]
            sl = slice(g * _GROUP, (g + 1) * _GROUP)
            sg = s_ref[:, sl]
            am = jnp.min(jnp.where(sg == m, lio, _GROUP), axis=1,
                         keepdims=True)
            ci = am + (kb * _BK + g * _GROUP)
            s_ref[:, sl] = jnp.where(lio == am, neg_inf, sg)
            v = v_ref[...]
            i = i_ref[...]
            cnt = jnp.sum(
                ((v > m) | ((v == m) & (i < ci))).astype(jnp.int32),
                axis=1, keepdims=True)
            vsh = jnp.concatenate([v[:, :1], v[:, :_TOPK - 1]], axis=1)
            ish = jnp.concatenate([i[:, :1], i[:, :_TOPK - 1]], axis=1)
            v_ref[...] = jnp.where(tio < cnt, v,
                                   jnp.where(tio == cnt, m, vsh))
            i_ref[...] = jnp.where(tio < cnt, i,
                                   jnp.where(tio == cnt, ci, ish))
        _group_max()
        return _maybe_improves()

    jax.lax.while_loop(_cond, _body, _maybe_improves())

    @pl.when(kb == nk - 1)
    def _finalize():
        vals_ref[...] = v_ref[...]
        idxs_ref[...] = i_ref[...]


def kernel(queries, keys):
    qn, d = queries.shape
    kdim = keys.shape[0]
    nk = pl.cdiv(kdim, _BK)
    ngrp = _BK // _GROUP
    vals, idxs = pl.pallas_call(
        functools.partial(_topk_kernel, nk=nk, kdim=kdim),
        grid=(nk,),
        in_specs=[
            pl.BlockSpec((qn, d), lambda kb: (0, 0)),
            pl.BlockSpec((_BK, d), lambda kb: (kb, 0)),
        ],
        out_specs=[
            pl.BlockSpec((qn, _TOPK), lambda kb: (0, 0)),
            pl.BlockSpec((qn, _TOPK), lambda kb: (0, 0)),
        ],
        out_shape=[
            jax.ShapeDtypeStruct((qn, _TOPK), jnp.float32),
            jax.ShapeDtypeStruct((qn, _TOPK), jnp.int32),
        ],
        scratch_shapes=[
            pltpu.VMEM((qn, _BK), jnp.float32),
            pltpu.VMEM((qn, _TOPK), jnp.float32),
            pltpu.VMEM((qn, _TOPK), jnp.int32),
            pltpu.VMEM((qn, ngrp), jnp.float32),
        ],
        compiler_params=pltpu.CompilerParams(
            dimension_semantics=("arbitrary",)),
    )(queries, keys)
    return vals, idxs


# speedup vs baseline: 3.0004x; 1.2831x over previous
import functools

import jax
import jax.numpy as jnp
from jax.experimental import pallas as pl
from jax.experimental.pallas import tpu as pltpu

_TOPK = 16
_TEMP = 10.0
_BK = 2048
_GROUP = 1024
_IMAX = 2147483647


def _topk_kernel(q_ref, k_ref, vals_ref, idxs_ref,
                 s_ref, v_ref, i_ref, gv_ref, *, nk, kdim):
    kb = pl.program_id(0)
    q = q_ref.shape[0]
    ngrp = _BK // _GROUP
    neg_inf = jnp.float32(-jnp.inf)

    @pl.when(kb == 0)
    def _init():
        v_ref[...] = jnp.full((q, _TOPK), neg_inf, jnp.float32)
        i_ref[...] = jnp.full((q, _TOPK), _IMAX, jnp.int32)

    s = jax.lax.dot_general(
        q_ref[...], k_ref[...], (((1,), (1,)), ((), ())),
        preferred_element_type=jnp.float32) * _TEMP
    s_ref[...] = s

    @pl.when(kb == nk - 1)
    def _mask_tail():
        rem = kdim - kb * _BK
        col = jax.lax.broadcasted_iota(jnp.int32, (q, _BK), 1)
        s_ref[...] = jnp.where(col < rem, s_ref[...], neg_inf)

    tio = jax.lax.broadcasted_iota(jnp.int32, (q, _TOPK), 1)
    lio = jax.lax.broadcasted_iota(jnp.int32, (q, _GROUP), 1)

    def _group_max():
        gv_ref[...] = jnp.concatenate(
            [jnp.max(s_ref[:, g * _GROUP:(g + 1) * _GROUP], axis=1,
                     keepdims=True) for g in range(ngrp)], axis=1)

    def _maybe_improves():
        return jnp.any(gv_ref[...] >= v_ref[:, _TOPK - 1:_TOPK])

    _group_max()

    def _cond(cont):
        return cont

    def _body(cont):
        gv = gv_ref[...]
        for g in range(ngrp):
            m = gv[:, g:g + 1]
            sl = slice(g * _GROUP, (g + 1) * _GROUP)
            sg = s_ref[:, sl]
            am = jnp.min(jnp.where(sg == m, lio, _GROUP), axis=1,
                         keepdims=True)
            ci = am + (kb * _BK + g * _GROUP)
            s_ref[:, sl] = jnp.where(lio == am, neg_inf, sg)
            v = v_ref[...]
            i = i_ref[...]
            cnt = jnp.sum(
                ((v > m) | ((v == m) & (i < ci))).astype(jnp.int32),
                axis=1, keepdims=True)
            vsh = jnp.concatenate([v[:, :1], v[:, :_TOPK - 1]], axis=1)
            ish = jnp.concatenate([i[:, :1], i[:, :_TOPK - 1]], axis=1)
            v_ref[...] = jnp.where(tio < cnt, v,
                                   jnp.where(tio == cnt, m, vsh))
            i_ref[...] = jnp.where(tio < cnt, i,
                                   jnp.where(tio == cnt, ci, ish))
        _group_max()
        return _maybe_improves()

    jax.lax.while_loop(_cond, _body, _maybe_improves())

    @pl.when(kb == nk - 1)
    def _finalize():
        vals_ref[...] = v_ref[...]
        idxs_ref[...] = i_ref[...]


def kernel(queries, keys):
    qn, d = queries.shape
    kdim = keys.shape[0]
    nk = pl.cdiv(kdim, _BK)
    ngrp = _BK // _GROUP
    vals, idxs = pl.pallas_call(
        functools.partial(_topk_kernel, nk=nk, kdim=kdim),
        grid=(nk,),
        in_specs=[
            pl.BlockSpec((qn, d), lambda kb: (0, 0)),
            pl.BlockSpec((_BK, d), lambda kb: (kb, 0)),
        ],
        out_specs=[
            pl.BlockSpec((qn, _TOPK), lambda kb: (0, 0)),
            pl.BlockSpec((qn, _TOPK), lambda kb: (0, 0)),
        ],
        out_shape=[
            jax.ShapeDtypeStruct((qn, _TOPK), jnp.float32),
            jax.ShapeDtypeStruct((qn, _TOPK), jnp.int32),
        ],
        scratch_shapes=[
            pltpu.VMEM((qn, _BK), jnp.float32),
            pltpu.VMEM((qn, _TOPK), jnp.float32),
            pltpu.VMEM((qn, _TOPK), jnp.int32),
            pltpu.VMEM((qn, ngrp), jnp.float32),
        ],
        compiler_params=pltpu.CompilerParams(
            dimension_semantics=("arbitrary",)),
    )(queries, keys)
    return vals, idxs


# speedup vs baseline: 3.2815x; 1.0937x over previous
import functools

import jax
import jax.numpy as jnp
from jax.experimental import pallas as pl
from jax.experimental.pallas import tpu as pltpu

_TOPK = 16
_TEMP = 10.0
_BK = 2048
_GROUP = 2048
_IMAX = 2147483647


def _topk_kernel(q_ref, k_ref, vals_ref, idxs_ref,
                 s_ref, v_ref, i_ref, gv_ref, *, nk, kdim):
    kb = pl.program_id(0)
    q = q_ref.shape[0]
    ngrp = _BK // _GROUP
    neg_inf = jnp.float32(-jnp.inf)

    @pl.when(kb == 0)
    def _init():
        v_ref[...] = jnp.full((q, _TOPK), neg_inf, jnp.float32)
        i_ref[...] = jnp.full((q, _TOPK), _IMAX, jnp.int32)

    s = jax.lax.dot_general(
        q_ref[...], k_ref[...], (((1,), (1,)), ((), ())),
        preferred_element_type=jnp.float32) * _TEMP
    s_ref[...] = s

    @pl.when(kb == nk - 1)
    def _mask_tail():
        rem = kdim - kb * _BK
        col = jax.lax.broadcasted_iota(jnp.int32, (q, _BK), 1)
        s_ref[...] = jnp.where(col < rem, s_ref[...], neg_inf)

    tio = jax.lax.broadcasted_iota(jnp.int32, (q, _TOPK), 1)
    lio = jax.lax.broadcasted_iota(jnp.int32, (q, _GROUP), 1)

    def _group_max():
        gv_ref[...] = jnp.concatenate(
            [jnp.max(s_ref[:, g * _GROUP:(g + 1) * _GROUP], axis=1,
                     keepdims=True) for g in range(ngrp)], axis=1)

    def _maybe_improves():
        return jnp.any(gv_ref[...] >= v_ref[:, _TOPK - 1:_TOPK])

    _group_max()

    def _cond(cont):
        return cont

    def _body(cont):
        gv = gv_ref[...]
        for g in range(ngrp):
            m = gv[:, g:g + 1]
            sl = slice(g * _GROUP, (g + 1) * _GROUP)
            sg = s_ref[:, sl]
            am = jnp.min(jnp.where(sg == m, lio, _GROUP), axis=1,
                         keepdims=True)
            ci = am + (kb * _BK + g * _GROUP)
            s_ref[:, sl] = jnp.where(lio == am, neg_inf, sg)
            v = v_ref[...]
            i = i_ref[...]
            cnt = jnp.sum(
                ((v > m) | ((v == m) & (i < ci))).astype(jnp.int32),
                axis=1, keepdims=True)
            vsh = jnp.concatenate([v[:, :1], v[:, :_TOPK - 1]], axis=1)
            ish = jnp.concatenate([i[:, :1], i[:, :_TOPK - 1]], axis=1)
            v_ref[...] = jnp.where(tio < cnt, v,
                                   jnp.where(tio == cnt, m, vsh))
            i_ref[...] = jnp.where(tio < cnt, i,
                                   jnp.where(tio == cnt, ci, ish))
        _group_max()
        return _maybe_improves()

    jax.lax.while_loop(_cond, _body, _maybe_improves())

    @pl.when(kb == nk - 1)
    def _finalize():
        vals_ref[...] = v_ref[...]
        idxs_ref[...] = i_ref[...]


def kernel(queries, keys):
    qn, d = queries.shape
    kdim = keys.shape[0]
    nk = pl.cdiv(kdim, _BK)
    ngrp = _BK // _GROUP
    vals, idxs = pl.pallas_call(
        functools.partial(_topk_kernel, nk=nk, kdim=kdim),
        grid=(nk,),
        in_specs=[
            pl.BlockSpec((qn, d), lambda kb: (0, 0)),
            pl.BlockSpec((_BK, d), lambda kb: (kb, 0)),
        ],
        out_specs=[
            pl.BlockSpec((qn, _TOPK), lambda kb: (0, 0)),
            pl.BlockSpec((qn, _TOPK), lambda kb: (0, 0)),
        ],
        out_shape=[
            jax.ShapeDtypeStruct((qn, _TOPK), jnp.float32),
            jax.ShapeDtypeStruct((qn, _TOPK), jnp.int32),
        ],
        scratch_shapes=[
            pltpu.VMEM((qn, _BK), jnp.float32),
            pltpu.VMEM((qn, _TOPK), jnp.float32),
            pltpu.VMEM((qn, _TOPK), jnp.int32),
            pltpu.VMEM((qn, ngrp), jnp.float32),
        ],
        compiler_params=pltpu.CompilerParams(
            dimension_semantics=("arbitrary",)),
    )(queries, keys)
    return vals, idxs
